# baseline (speedup 1.0000x reference)
import functools
import numpy as np
import jax
import jax.numpy as jnp
from jax import lax
from jax.experimental import pallas as pl
from jax.experimental.pallas import tpu as pltpu
from jax.experimental.pallas import tpu_sc as plsc

_LAYER_IDX = 6
_NUM_LAYERS = 12
_DIM = 2048

_NSTRIPE = 16
_R = 128
_NQ = 4
_RING = _NQ * _R
_MAXOFF = 128


def _shift_offsets(v_len, d_v):
    i = np.arange(d_v, dtype=np.float64)
    e = (_LAYER_IDX * _DIM + i) / (_NUM_LAYERS * _DIM)
    return np.ceil(np.power(float(v_len), e)).astype(np.int64)


@functools.lru_cache(maxsize=None)
def _build(B, N, D):
    off = _shift_offsets(N, D)
    assert off.min() >= 1 and off.max() <= _MAXOFF
    delta_np = (_MAXOFF - off).astype(np.int32)
    CPW = D // _NSTRIPE
    NC16 = CPW // 16
    NBLK = N // _R
    LINMASK = _RING * CPW - 1
    assert N % _RING == 0 and D % _NSTRIPE == 0 and CPW == 128
    assert B % 2 == 0 and NBLK % 2 == 0

    mesh = plsc.VectorSubcoreMesh(core_axis_name="c", subcore_axis_name="s")

    @functools.partial(
        pl.kernel,
        out_type=jax.ShapeDtypeStruct((B, N, D), jnp.float32),
        mesh=mesh,
        compiler_params=pltpu.CompilerParams(needs_layout_passes=False),
        scratch_types=[
            pltpu.VMEM((CPW,), jnp.int32),
            pltpu.VMEM((_RING, CPW), jnp.float32),
            pltpu.VMEM((2, _R, CPW), jnp.float32),
            pltpu.SemaphoreType.DMA,
            pltpu.SemaphoreType.DMA,
            pltpu.SemaphoreType.DMA,
            pltpu.SemaphoreType.DMA,
            pltpu.SemaphoreType.DMA,
            pltpu.SemaphoreType.DMA,
        ],
    )
    def swd(v_hbm, delta_hbm, out_hbm, delta_v, ring_v, dst_v,
            semi0, semi1, semi2, semi3, semo0, semo1):
        wid = lax.axis_index("s") * 2 + lax.axis_index("c")
        stripe = wid // 2
        half = wid % 2
        c0 = stripe * CPW
        pltpu.sync_copy(delta_hbm.at[pl.ds(c0, CPW)], delta_v)
        iota = lax.iota(jnp.int32, 16)
        zero16 = iota * 0
        bneg = [((delta_v[pl.ds(c * 16, 16)] - _MAXOFF) << 7)
                + (c * 16 + iota) for c in range(NC16)]
        bneg2 = [b + CPW for b in bneg]
        semi = [semi0, semi1, semi2, semi3]
        semo = [semo0, semo1]

        def start_stage(b, s, q):
            pltpu.async_copy(
                v_hbm.at[b, pl.ds(s * _R, _R), pl.ds(c0, CPW)],
                ring_v.at[pl.ds(q * _R, _R)], semi[q])

        def start_tail(b):
            pltpu.async_copy(
                v_hbm.at[b, pl.ds(N - _R, _R), pl.ds(c0, CPW)],
                ring_v.at[pl.ds((_NQ - 1) * _R, _R)], semi[_NQ - 1])

        def wait_stage(q):
            pltpu.make_async_copy(
                v_hbm.at[0, pl.ds(0, _R), pl.ds(c0, CPW)],
                ring_v.at[pl.ds(q * _R, _R)], semi[q]).wait()

        def start_out(b, jj, k):
            pltpu.async_copy(
                dst_v.at[k], out_hbm.at[b, pl.ds(jj * _R, _R), pl.ds(c0, CPW)],
                semo[k])

        def wait_out(k):
            pltpu.make_async_copy(
                dst_v.at[k], out_hbm.at[0, pl.ds(0, _R), pl.ds(c0, CPW)],
                semo[k]).wait()

        def compute(jj, k):
            dbuf = dst_v.at[k]
            jbase = jj * _R

            @plsc.parallel_loop(0, _R // 2, unroll=2)
            def pair(p):
                r = 2 * p
                g = (jbase + r) << 7
                los = [plsc.load_gather(ring_v,
                                        [zero16, (g + bneg[c]) & LINMASK])
                       for c in range(NC16)]
                his = [plsc.load_gather(ring_v,
                                        [zero16, (g + bneg2[c]) & LINMASK])
                       for c in range(NC16)]
                for c in range(NC16):
                    dbuf[r, pl.ds(c * 16, 16)] = jnp.minimum(los[c], his[c])
                    dbuf[r + 1, pl.ds(c * 16, 16)] = jnp.maximum(los[c], his[c])

        for bi in range(B // 2):
            b = half * (B // 2) + bi
            start_tail(b)
            start_stage(b, 0, 0)
            start_stage(b, 1, 1)

            @pl.loop(0, NBLK, step=_NQ)
            def slot(j):
                for k in range(_NQ):
                    jj = j + k

                    @pl.when(jj + 2 < NBLK)
                    def _():
                        start_stage(b, jj + 2, (k + 2) % _NQ)

                    if k == 0:
                        @pl.when(jj == 0)
                        def _():
                            wait_stage(_NQ - 1)

                    wait_stage(k)

                    if bi == 0:
                        @pl.when(jj >= 2)
                        def _():
                            wait_out(k % 2)
                    else:
                        wait_out(k % 2)

                    compute(jj, k % 2)
                    start_out(b, jj, k % 2)

        for k in range(2):
            wait_out(k)

    def call(v, delta):
        return swd(v, delta)

    return call, jnp.asarray(delta_np)


def kernel(v):
    B, N, D = v.shape
    call, delta = _build(B, N, D)
    return call(v, delta)


# speedup vs baseline: 1.0786x; 1.0786x over previous
import functools
import numpy as np
import jax
import jax.numpy as jnp
from jax import lax
from jax.experimental import pallas as pl
from jax.experimental.pallas import tpu as pltpu
from jax.experimental.pallas import tpu_sc as plsc

_LAYER_IDX = 6
_NUM_LAYERS = 12
_DIM = 2048

_NSTRIPE = 16
_R = 128
_NQ = 4
_RING = _NQ * _R
_MAXOFF = 128


def _shift_offsets(v_len, d_v):
    i = np.arange(d_v, dtype=np.float64)
    e = (_LAYER_IDX * _DIM + i) / (_NUM_LAYERS * _DIM)
    return np.ceil(np.power(float(v_len), e)).astype(np.int64)


@functools.lru_cache(maxsize=None)
def _build(B, N, D):
    off = _shift_offsets(N, D)
    assert off.min() >= 1 and off.max() <= _MAXOFF
    delta_np = (_MAXOFF - off).astype(np.int32)
    CPW = D // _NSTRIPE
    NC16 = CPW // 16
    NBLK = N // _R
    LINMASK = _RING * CPW - 1
    assert N % _RING == 0 and D % _NSTRIPE == 0 and CPW == 128
    assert B % 2 == 0 and NBLK % 2 == 0

    mesh = plsc.VectorSubcoreMesh(core_axis_name="c", subcore_axis_name="s")

    @functools.partial(
        pl.kernel,
        out_type=jax.ShapeDtypeStruct((B, N, D), jnp.float32),
        mesh=mesh,
        compiler_params=pltpu.CompilerParams(needs_layout_passes=False),
        scratch_types=[
            pltpu.VMEM((CPW,), jnp.int32),
            pltpu.VMEM((_RING, CPW), jnp.float32),
            pltpu.VMEM((2, _R, CPW), jnp.float32),
            pltpu.SemaphoreType.DMA,
            pltpu.SemaphoreType.DMA,
            pltpu.SemaphoreType.DMA,
            pltpu.SemaphoreType.DMA,
            pltpu.SemaphoreType.DMA,
            pltpu.SemaphoreType.DMA,
        ],
    )
    def swd(v_hbm, delta_hbm, out_hbm, delta_v, ring_v, dst_v,
            semi0, semi1, semi2, semi3, semo0, semo1):
        wid = lax.axis_index("s") * 2 + lax.axis_index("c")
        stripe = wid // 2
        half = wid % 2
        c0 = stripe * CPW
        pltpu.sync_copy(delta_hbm.at[pl.ds(c0, CPW)], delta_v)
        iota = lax.iota(jnp.int32, 16)
        zero16 = iota * 0
        bneg = [((delta_v[pl.ds(c * 16, 16)] - _MAXOFF) << 7)
                + (c * 16 + iota) for c in range(NC16)]
        semi = [semi0, semi1, semi2, semi3]
        semo = [semo0, semo1]

        def start_stage(b, s, q):
            pltpu.async_copy(
                v_hbm.at[b, pl.ds(s * _R, _R), pl.ds(c0, CPW)],
                ring_v.at[pl.ds(q * _R, _R)], semi[q])

        def start_tail(b):
            pltpu.async_copy(
                v_hbm.at[b, pl.ds(N - _R, _R), pl.ds(c0, CPW)],
                ring_v.at[pl.ds((_NQ - 1) * _R, _R)], semi[_NQ - 1])

        def wait_stage(q):
            pltpu.make_async_copy(
                v_hbm.at[0, pl.ds(0, _R), pl.ds(c0, CPW)],
                ring_v.at[pl.ds(q * _R, _R)], semi[q]).wait()

        def start_out(b, jj, k):
            pltpu.async_copy(
                dst_v.at[k], out_hbm.at[b, pl.ds(jj * _R, _R), pl.ds(c0, CPW)],
                semo[k])

        def wait_out(k):
            pltpu.make_async_copy(
                dst_v.at[k], out_hbm.at[0, pl.ds(0, _R), pl.ds(c0, CPW)],
                semo[k]).wait()

        def compute(q, kd):
            dbuf = dst_v.at[kd]

            @plsc.parallel_loop(0, _R // 2, unroll=2)
            def pair(p):
                rs = ((2 * p) << 7) + (q << 14)
                ts = [bneg[c] + rs for c in range(NC16)]
                if q == 0:
                    lo_lin = [t & LINMASK for t in ts]
                    hi_lin = [(t + CPW) & LINMASK for t in ts]
                else:
                    lo_lin = ts
                    hi_lin = [t + CPW for t in ts]
                los = [plsc.load_gather(ring_v, [zero16, lo_lin[c]])
                       for c in range(NC16)]
                his = [plsc.load_gather(ring_v, [zero16, hi_lin[c]])
                       for c in range(NC16)]
                r = 2 * p
                for c in range(NC16):
                    dbuf[r, pl.ds(c * 16, 16)] = jnp.minimum(los[c], his[c])
                    dbuf[r + 1, pl.ds(c * 16, 16)] = jnp.maximum(los[c], his[c])

        for bi in range(B // 2):
            b = half * (B // 2) + bi
            start_tail(b)
            start_stage(b, 0, 0)
            start_stage(b, 1, 1)

            @pl.loop(0, NBLK, step=_NQ)
            def slot(j):
                for k in range(_NQ):
                    jj = j + k

                    @pl.when(jj + 2 < NBLK)
                    def _():
                        start_stage(b, jj + 2, (k + 2) % _NQ)

                    if k == 0:
                        @pl.when(jj == 0)
                        def _():
                            wait_stage(_NQ - 1)

                    wait_stage(k)

                    if bi == 0:
                        @pl.when(jj >= 2)
                        def _():
                            wait_out(k % 2)
                    else:
                        wait_out(k % 2)

                    compute(k, k % 2)
                    start_out(b, jj, k % 2)

        for k in range(2):
            wait_out(k)

    def call(v, delta):
        return swd(v, delta)

    return call, jnp.asarray(delta_np)


def kernel(v):
    B, N, D = v.shape
    call, delta = _build(B, N, D)
    return call(v, delta)


# speedup vs baseline: 1.1041x; 1.0237x over previous
import functools
import numpy as np
import jax
import jax.numpy as jnp
from jax import lax
from jax.experimental import pallas as pl
from jax.experimental.pallas import tpu as pltpu
from jax.experimental.pallas import tpu_sc as plsc

_LAYER_IDX = 6
_NUM_LAYERS = 12
_DIM = 2048

_NSTRIPE = 16
_R = 128
_NQ = 4
_RING = _NQ * _R
_MAXOFF = 128


def _shift_offsets(v_len, d_v):
    i = np.arange(d_v, dtype=np.float64)
    e = (_LAYER_IDX * _DIM + i) / (_NUM_LAYERS * _DIM)
    return np.ceil(np.power(float(v_len), e)).astype(np.int64)


@functools.lru_cache(maxsize=None)
def _build(B, N, D):
    off = _shift_offsets(N, D)
    assert off.min() >= 1 and off.max() <= _MAXOFF
    delta_np = (_MAXOFF - off).astype(np.int32)
    CPW = D // _NSTRIPE
    NC16 = CPW // 16
    NBLK = N // _R
    LINMASK = _RING * CPW - 1
    assert N % _RING == 0 and D % _NSTRIPE == 0 and CPW == 128
    assert B % 2 == 0 and NBLK % 2 == 0

    mesh = plsc.VectorSubcoreMesh(core_axis_name="c", subcore_axis_name="s")

    @functools.partial(
        pl.kernel,
        out_type=jax.ShapeDtypeStruct((B, N, D), jnp.float32),
        mesh=mesh,
        compiler_params=pltpu.CompilerParams(needs_layout_passes=False),
        scratch_types=[
            pltpu.VMEM((CPW,), jnp.int32),
            pltpu.VMEM((_RING, CPW), jnp.float32),
            pltpu.VMEM((2, _R, CPW), jnp.float32),
            pltpu.SemaphoreType.DMA,
            pltpu.SemaphoreType.DMA,
            pltpu.SemaphoreType.DMA,
            pltpu.SemaphoreType.DMA,
            pltpu.SemaphoreType.DMA,
            pltpu.SemaphoreType.DMA,
        ],
    )
    def swd(v_hbm, delta_hbm, out_hbm, delta_v, ring_v, dst_v,
            semi0, semi1, semi2, semi3, semo0, semo1):
        wid = lax.axis_index("s") * 2 + lax.axis_index("c")
        stripe = wid // 2
        half = wid % 2
        c0 = stripe * CPW
        pltpu.sync_copy(delta_hbm.at[pl.ds(c0, CPW)], delta_v)
        iota = lax.iota(jnp.int32, 16)
        zero16 = iota * 0
        bneg = [((delta_v[pl.ds(c * 16, 16)] - _MAXOFF) << 7)
                + (c * 16 + iota) for c in range(NC16)]
        semi = [semi0, semi1, semi2, semi3]
        semo = [semo0, semo1]

        def start_stage(b, s, q):
            pltpu.async_copy(
                v_hbm.at[b, pl.ds(s * _R, _R), pl.ds(c0, CPW)],
                ring_v.at[pl.ds(q * _R, _R)], semi[q])

        def start_tail(b):
            pltpu.async_copy(
                v_hbm.at[b, pl.ds(N - _R, _R), pl.ds(c0, CPW)],
                ring_v.at[pl.ds((_NQ - 1) * _R, _R)], semi[_NQ - 1])

        def wait_stage(q):
            pltpu.make_async_copy(
                v_hbm.at[0, pl.ds(0, _R), pl.ds(c0, CPW)],
                ring_v.at[pl.ds(q * _R, _R)], semi[q]).wait()

        def start_out(b, jj, k):
            pltpu.async_copy(
                dst_v.at[k], out_hbm.at[b, pl.ds(jj * _R, _R), pl.ds(c0, CPW)],
                semo[k])

        def wait_out(k):
            pltpu.make_async_copy(
                dst_v.at[k], out_hbm.at[0, pl.ds(0, _R), pl.ds(c0, CPW)],
                semo[k]).wait()

        def compute(q, kd):
            dbuf = dst_v.at[kd]

            @plsc.parallel_loop(0, _R // 2, unroll=1)
            def pair(p):
                rs = ((2 * p) << 7) + (q << 14)
                ts = [bneg[c] + rs for c in range(NC16)]
                if q == 0:
                    lo_lin = [t & LINMASK for t in ts]
                    hi_lin = [(t + CPW) & LINMASK for t in ts]
                else:
                    lo_lin = ts
                    hi_lin = [t + CPW for t in ts]
                los = [plsc.load_gather(ring_v, [zero16, lo_lin[c]])
                       for c in range(NC16)]
                his = [plsc.load_gather(ring_v, [zero16, hi_lin[c]])
                       for c in range(NC16)]
                r = 2 * p
                for c in range(NC16):
                    dbuf[r, pl.ds(c * 16, 16)] = jnp.minimum(los[c], his[c])
                    dbuf[r + 1, pl.ds(c * 16, 16)] = jnp.maximum(los[c], his[c])

        for bi in range(B // 2):
            b = half * (B // 2) + bi
            start_tail(b)
            start_stage(b, 0, 0)
            start_stage(b, 1, 1)

            @pl.loop(0, NBLK, step=_NQ)
            def slot(j):
                for k in range(_NQ):
                    jj = j + k

                    @pl.when(jj + 2 < NBLK)
                    def _():
                        start_stage(b, jj + 2, (k + 2) % _NQ)

                    if k == 0:
                        @pl.when(jj == 0)
                        def _():
                            wait_stage(_NQ - 1)

                    wait_stage(k)

                    if bi == 0:
                        @pl.when(jj >= 2)
                        def _():
                            wait_out(k % 2)
                    else:
                        wait_out(k % 2)

                    compute(k, k % 2)
                    start_out(b, jj, k % 2)

        for k in range(2):
            wait_out(k)

    def call(v, delta):
        return swd(v, delta)

    return call, jnp.asarray(delta_np)


def kernel(v):
    B, N, D = v.shape
    call, delta = _build(B, N, D)
    return call(v, delta)
